# baseline (speedup 1.0000x reference)
import functools

import jax
import jax.numpy as jnp
from jax import lax
from jax.experimental import pallas as pl
from jax.experimental.pallas import tpu as pltpu
from jax.experimental.pallas import tpu_sc as plsc

_BATCH = 1024
_SEQ = 200
_EMBED = 64

_NC = 2
_NS = 16
_NW = _NC * _NS

_NBLK = _SEQ * (_BATCH // 128)
_BLK_PER_W = _NBLK // _NW
_NB = 3
_LOOK = 2
_N_MAIN = 48


def _gather_kernel(tbl_hbm, idx_hbm, out_hbm, idx_v, w_v, rows_v, outb_v,
                   isem, gsem, osem):
    wid = lax.axis_index("s") * _NC + lax.axis_index("c")
    c0 = wid * _BLK_PER_W
    iota16 = lax.iota(jnp.int32, 16)

    def blk_pos(j):
        c = c0 + j
        return c // 8, lax.rem(c, 8)

    for j in range(_BLK_PER_W):
        s, tb = blk_pos(j)
        pltpu.make_async_copy(
            idx_hbm.at[s, pl.ds(tb * 128, 128)], idx_v.at[j], isem
        ).start()
    for j in range(_BLK_PER_W):
        s, tb = blk_pos(j)
        pltpu.make_async_copy(
            idx_hbm.at[s, pl.ds(tb * 128, 128)], idx_v.at[j], isem
        ).wait()

    def prep_gather(j, slot):
        for g in range(8):
            v = idx_v[j, pl.ds(g * 16, 16)]
            w_v[slot, pl.ds(g * 16, 16)] = lax.shift_right_logical(v, 1)
        pltpu.make_async_copy(
            tbl_hbm.at[w_v.at[slot]], rows_v.at[slot], gsem.at[slot]
        ).start()

    def wait_gather(slot):
        pltpu.make_async_copy(
            tbl_hbm.at[w_v.at[slot]], rows_v.at[slot], gsem.at[slot]
        ).wait()

    def fire_out(j, slot):
        s, tb = blk_pos(j)
        pltpu.make_async_copy(
            outb_v.at[slot], out_hbm.at[s, :, pl.ds(tb * 128, 128)],
            osem.at[slot],
        ).start()

    def wait_out(j, slot):
        s, tb = blk_pos(j)
        pltpu.make_async_copy(
            outb_v.at[slot], out_hbm.at[s, :, pl.ds(tb * 128, 128)],
            osem.at[slot],
        ).wait()

    def transpose_block(j, slot):
        def ebody(e0, carry):
            for g in range(8):
                v = idx_v[j, pl.ds(g * 16, 16)]
                base_col = lax.shift_left(lax.bitwise_and(v, 1), 6)
                row_vec = iota16 + g * 16
                for ei in range(4):
                    e = e0 * 4 + ei
                    val = plsc.load_gather(
                        rows_v.at[slot], [row_vec, base_col + e]
                    )
                    outb_v[slot, e, pl.ds(g * 16, 16)] = val
            return carry

        lax.fori_loop(0, 16, ebody, 0)

    for k in range(_LOOK):
        prep_gather(k, k)

    def body(gidx, carry):
        for b in range(_NB):
            j = gidx * _NB + b
            wait_gather(b)

            @pl.when(j >= _NB)
            def _():
                wait_out(j - _NB, b)

            transpose_block(j, b)
            fire_out(j, b)

            @pl.when(j + _LOOK < _BLK_PER_W)
            def _():
                prep_gather(j + _LOOK, (b + _LOOK) % _NB)

        return carry

    lax.fori_loop(0, _N_MAIN // _NB, body, 0)

    for j in (48, 49):
        slot = j % _NB
        wait_gather(slot)
        wait_out(j - _NB, slot)
        transpose_block(j, slot)
        fire_out(j, slot)

    for j in (47, 48, 49):
        wait_out(j, j % _NB)


_NTC = 7813
_TC_PER_W = 245


def _transpose_kernel(tbl_hbm, tail_hbm, out_hbm, stg_v, tout_v, isem, osem):
    wid = lax.axis_index("s") * _NC + lax.axis_index("c")
    iota16 = lax.iota(jnp.int32, 16)

    def col_of(k):
        return wid + k * _NW

    def fire_in(k, slot):
        c = col_of(k)

        @pl.when(c < _NTC - 1)
        def _():
            pltpu.make_async_copy(
                tbl_hbm.at[:, pl.ds(c * 128, 128)], stg_v.at[slot],
                isem.at[slot],
            ).start()

        @pl.when(c == _NTC - 1)
        def _():
            pltpu.make_async_copy(
                tail_hbm, stg_v.at[slot], isem.at[slot]
            ).start()

    def wait_in(k, slot):
        c = col_of(k)

        @pl.when(c < _NTC - 1)
        def _():
            pltpu.make_async_copy(
                tbl_hbm.at[:, pl.ds(c * 128, 128)], stg_v.at[slot],
                isem.at[slot],
            ).wait()

        @pl.when(c == _NTC - 1)
        def _():
            pltpu.make_async_copy(
                tail_hbm, stg_v.at[slot], isem.at[slot]
            ).wait()

    def transpose_col(k, slot):
        def wbody(wr0, carry):
            for wi in range(4):
                wr = wr0 * 4 + wi
                for g in range(8):
                    row_vec = iota16 + 16 * (g % 4)
                    col_vec = iota16 * 0 + (2 * wr + g // 4)
                    val = plsc.load_gather(stg_v.at[slot], [row_vec, col_vec])
                    tout_v[slot, wr, pl.ds(g * 16, 16)] = val
            return carry

        lax.fori_loop(0, 16, wbody, 0)

    def fire_out(k, slot):
        c = col_of(k)

        @pl.when(c < _NTC - 1)
        def _():
            pltpu.make_async_copy(
                tout_v.at[slot], out_hbm.at[pl.ds(c * 64, 64)], osem.at[slot]
            ).start()

        @pl.when(c == _NTC - 1)
        def _():
            pltpu.make_async_copy(
                tout_v.at[slot, pl.ds(0, 32)],
                out_hbm.at[pl.ds(c * 64, 32)], osem.at[slot],
            ).start()

    def wait_out(k, slot):
        c = col_of(k)

        @pl.when(c < _NTC - 1)
        def _():
            pltpu.make_async_copy(
                tout_v.at[slot], out_hbm.at[pl.ds(c * 64, 64)], osem.at[slot]
            ).wait()

        @pl.when(c == _NTC - 1)
        def _():
            pltpu.make_async_copy(
                tout_v.at[slot, pl.ds(0, 32)],
                out_hbm.at[pl.ds(c * 64, 32)], osem.at[slot],
            ).wait()

    def valid(k):
        return col_of(k) < _NTC

    def step(k, slot):
        @pl.when(valid(k))
        def _():
            wait_in(k, slot)

        @pl.when(valid(k + 1))
        def _():
            fire_in(k + 1, slot ^ 1)

        @pl.when(valid(k))
        def _():
            @pl.when(k >= 2)
            def _():
                wait_out(k - 2, slot)

            transpose_col(k, slot)
            fire_out(k, slot)

    @pl.when(valid(0))
    def _():
        fire_in(0, 0)

    def body(m, carry):
        step(2 * m, 0)
        step(2 * m + 1, 1)
        return carry

    lax.fori_loop(0, (_TC_PER_W - 1) // 2, body, 0)
    step(jnp.int32(_TC_PER_W - 1), (_TC_PER_W - 1) % 2)

    for k in (_TC_PER_W - 2, _TC_PER_W - 1):
        @pl.when(valid(k))
        def _():
            wait_out(k, k % 2)


@jax.jit
def _transpose(tbl_t, tail_t):
    mesh = plsc.VectorSubcoreMesh(core_axis_name="c", subcore_axis_name="s")
    run = functools.partial(
        pl.kernel,
        mesh=mesh,
        out_type=jax.ShapeDtypeStruct((500000, 128), jnp.float32),
        scratch_types=[
            pltpu.VMEM((2, _EMBED, 128), jnp.float32),
            pltpu.VMEM((2, _EMBED, 128), jnp.float32),
            pltpu.SemaphoreType.DMA((2,)),
            pltpu.SemaphoreType.DMA((2,)),
        ],
        compiler_params=pltpu.CompilerParams(
            use_tc_tiling_on_sc=True, needs_layout_passes=False
        ),
    )(_transpose_kernel)
    return run(tbl_t, tail_t)


@jax.jit
def _gather(tbl2, idx_t):
    mesh = plsc.VectorSubcoreMesh(core_axis_name="c", subcore_axis_name="s")
    run = functools.partial(
        pl.kernel,
        mesh=mesh,
        out_type=jax.ShapeDtypeStruct((_SEQ, _EMBED, _BATCH), jnp.float32),
        scratch_types=[
            pltpu.VMEM((_BLK_PER_W, 128), jnp.int32),
            pltpu.VMEM((_NB, 128), jnp.int32),
            pltpu.VMEM((_NB, 128, 128), jnp.float32),
            pltpu.VMEM((_NB, _EMBED, 128), jnp.float32),
            pltpu.SemaphoreType.DMA,
            pltpu.SemaphoreType.DMA((_NB,)),
            pltpu.SemaphoreType.DMA((_NB,)),
        ],
        compiler_params=pltpu.CompilerParams(
            use_tc_tiling_on_sc=True, needs_layout_passes=False
        ),
    )(_gather_kernel)
    return run(tbl2, idx_t)


def kernel(input_seq, embedding_table):
    tbl_t = embedding_table.T
    tail_t = jnp.pad(tbl_t[:, (_NTC - 1) * 128:], ((0, 0), (0, 64)))
    tbl2 = _transpose(tbl_t, tail_t)
    idx_t = input_seq.astype(jnp.int32).T
    out_t = _gather(tbl2, idx_t)
    return jnp.transpose(out_t, (2, 0, 1))


# speedup vs baseline: 1.9174x; 1.9174x over previous
import functools

import jax
import jax.numpy as jnp
from jax import lax
from jax.experimental import pallas as pl
from jax.experimental.pallas import tpu as pltpu
from jax.experimental.pallas import tpu_sc as plsc

_BATCH = 1024
_SEQ = 200
_EMBED = 64

_NC = 2
_NS = 16
_NW = _NC * _NS

_NBLK = _SEQ * (_BATCH // 128)
_BLK_PER_W = _NBLK // _NW
_NB = 3
_LOOK = 2
_N_MAIN = 48


def _gather_kernel(tbl_hbm, idx_hbm, out_hbm, idx_v, w_v, rows_v, outb_v,
                   isem, gsem, osem):
    wid = lax.axis_index("s") * _NC + lax.axis_index("c")
    c0 = wid * _BLK_PER_W
    iota16 = lax.iota(jnp.int32, 16)

    def blk_pos(j):
        c = c0 + j
        return c // 8, lax.rem(c, 8)

    for j in range(_BLK_PER_W):
        s, tb = blk_pos(j)
        pltpu.make_async_copy(
            idx_hbm.at[s, pl.ds(tb * 128, 128)], idx_v.at[j], isem
        ).start()
    for j in range(_BLK_PER_W):
        s, tb = blk_pos(j)
        pltpu.make_async_copy(
            idx_hbm.at[s, pl.ds(tb * 128, 128)], idx_v.at[j], isem
        ).wait()

    def prep_gather(j, slot):
        for g in range(8):
            v = idx_v[j, pl.ds(g * 16, 16)]
            w_v[slot, pl.ds(g * 16, 16)] = lax.shift_right_logical(v, 1)
        pltpu.make_async_copy(
            tbl_hbm.at[w_v.at[slot]], rows_v.at[slot], gsem.at[slot]
        ).start()

    def wait_gather(slot):
        pltpu.make_async_copy(
            tbl_hbm.at[w_v.at[slot]], rows_v.at[slot], gsem.at[slot]
        ).wait()

    def fire_out(j, slot):
        s, tb = blk_pos(j)
        pltpu.make_async_copy(
            outb_v.at[slot], out_hbm.at[s, :, pl.ds(tb * 128, 128)],
            osem.at[slot],
        ).start()

    def wait_out(j, slot):
        s, tb = blk_pos(j)
        pltpu.make_async_copy(
            outb_v.at[slot], out_hbm.at[s, :, pl.ds(tb * 128, 128)],
            osem.at[slot],
        ).wait()

    row_vecs = [iota16 + g * 16 for g in range(8)]

    def transpose_block(j, slot):
        bases = []
        for g in range(8):
            v = idx_v[j, pl.ds(g * 16, 16)]
            bases.append(lax.shift_left(lax.bitwise_and(v, 1), 6))

        @plsc.parallel_loop(0, _EMBED, unroll=8)
        def ebody(e):
            for g in range(8):
                val = plsc.load_gather(
                    rows_v.at[slot], [row_vecs[g], bases[g] + e]
                )
                outb_v[slot, e, pl.ds(g * 16, 16)] = val

    for k in range(_LOOK):
        prep_gather(k, k)

    def body(gidx, carry):
        for b in range(_NB):
            j = gidx * _NB + b
            wait_gather(b)

            @pl.when(j >= _NB)
            def _():
                wait_out(j - _NB, b)

            transpose_block(j, b)
            fire_out(j, b)

            @pl.when(j + _LOOK < _BLK_PER_W)
            def _():
                prep_gather(j + _LOOK, (b + _LOOK) % _NB)

        return carry

    lax.fori_loop(0, _N_MAIN // _NB, body, 0)

    for j in (48, 49):
        slot = j % _NB
        wait_gather(slot)
        wait_out(j - _NB, slot)
        transpose_block(j, slot)
        fire_out(j, slot)

    for j in (47, 48, 49):
        wait_out(j, j % _NB)


_NTC = 7813
_TC_PER_W = 245


def _transpose_kernel(tbl_hbm, tail_hbm, out_hbm, stg_v, tout_v, isem, osem):
    wid = lax.axis_index("s") * _NC + lax.axis_index("c")
    iota16 = lax.iota(jnp.int32, 16)

    def col_of(k):
        return wid + k * _NW

    def fire_in(k, slot):
        c = col_of(k)

        @pl.when(c < _NTC - 1)
        def _():
            pltpu.make_async_copy(
                tbl_hbm.at[:, pl.ds(c * 128, 128)], stg_v.at[slot],
                isem.at[slot],
            ).start()

        @pl.when(c == _NTC - 1)
        def _():
            pltpu.make_async_copy(
                tail_hbm, stg_v.at[slot], isem.at[slot]
            ).start()

    def wait_in(k, slot):
        c = col_of(k)

        @pl.when(c < _NTC - 1)
        def _():
            pltpu.make_async_copy(
                tbl_hbm.at[:, pl.ds(c * 128, 128)], stg_v.at[slot],
                isem.at[slot],
            ).wait()

        @pl.when(c == _NTC - 1)
        def _():
            pltpu.make_async_copy(
                tail_hbm, stg_v.at[slot], isem.at[slot]
            ).wait()

    row_vecs = [iota16 + 16 * q for q in range(4)]

    def transpose_col(k, slot):
        @plsc.parallel_loop(0, _EMBED, unroll=8)
        def wbody(wr):
            cols = [jnp.full((16,), 2 * wr, jnp.int32),
                    jnp.full((16,), 2 * wr + 1, jnp.int32)]
            for g in range(8):
                val = plsc.load_gather(
                    stg_v.at[slot], [row_vecs[g % 4], cols[g // 4]]
                )
                tout_v[slot, wr, pl.ds(g * 16, 16)] = val

    def fire_out(k, slot):
        c = col_of(k)

        @pl.when(c < _NTC - 1)
        def _():
            pltpu.make_async_copy(
                tout_v.at[slot], out_hbm.at[pl.ds(c * 64, 64)], osem.at[slot]
            ).start()

        @pl.when(c == _NTC - 1)
        def _():
            pltpu.make_async_copy(
                tout_v.at[slot, pl.ds(0, 32)],
                out_hbm.at[pl.ds(c * 64, 32)], osem.at[slot],
            ).start()

    def wait_out(k, slot):
        c = col_of(k)

        @pl.when(c < _NTC - 1)
        def _():
            pltpu.make_async_copy(
                tout_v.at[slot], out_hbm.at[pl.ds(c * 64, 64)], osem.at[slot]
            ).wait()

        @pl.when(c == _NTC - 1)
        def _():
            pltpu.make_async_copy(
                tout_v.at[slot, pl.ds(0, 32)],
                out_hbm.at[pl.ds(c * 64, 32)], osem.at[slot],
            ).wait()

    def valid(k):
        return col_of(k) < _NTC

    def step(k, slot):
        @pl.when(valid(k))
        def _():
            wait_in(k, slot)

        @pl.when(valid(k + 1))
        def _():
            fire_in(k + 1, slot ^ 1)

        @pl.when(valid(k))
        def _():
            @pl.when(k >= 2)
            def _():
                wait_out(k - 2, slot)

            transpose_col(k, slot)
            fire_out(k, slot)

    @pl.when(valid(0))
    def _():
        fire_in(0, 0)

    def body(m, carry):
        step(2 * m, 0)
        step(2 * m + 1, 1)
        return carry

    lax.fori_loop(0, (_TC_PER_W - 1) // 2, body, 0)
    step(jnp.int32(_TC_PER_W - 1), (_TC_PER_W - 1) % 2)

    for k in (_TC_PER_W - 2, _TC_PER_W - 1):
        @pl.when(valid(k))
        def _():
            wait_out(k, k % 2)


@jax.jit
def _transpose(tbl_t, tail_t):
    mesh = plsc.VectorSubcoreMesh(core_axis_name="c", subcore_axis_name="s")
    run = functools.partial(
        pl.kernel,
        mesh=mesh,
        out_type=jax.ShapeDtypeStruct((500000, 128), jnp.float32),
        scratch_types=[
            pltpu.VMEM((2, _EMBED, 128), jnp.float32),
            pltpu.VMEM((2, _EMBED, 128), jnp.float32),
            pltpu.SemaphoreType.DMA((2,)),
            pltpu.SemaphoreType.DMA((2,)),
        ],
        compiler_params=pltpu.CompilerParams(
            use_tc_tiling_on_sc=True, needs_layout_passes=False
        ),
    )(_transpose_kernel)
    return run(tbl_t, tail_t)


@jax.jit
def _gather(tbl2, idx_t):
    mesh = plsc.VectorSubcoreMesh(core_axis_name="c", subcore_axis_name="s")
    run = functools.partial(
        pl.kernel,
        mesh=mesh,
        out_type=jax.ShapeDtypeStruct((_SEQ, _EMBED, _BATCH), jnp.float32),
        scratch_types=[
            pltpu.VMEM((_BLK_PER_W, 128), jnp.int32),
            pltpu.VMEM((_NB, 128), jnp.int32),
            pltpu.VMEM((_NB, 128, 128), jnp.float32),
            pltpu.VMEM((_NB, _EMBED, 128), jnp.float32),
            pltpu.SemaphoreType.DMA,
            pltpu.SemaphoreType.DMA((_NB,)),
            pltpu.SemaphoreType.DMA((_NB,)),
        ],
        compiler_params=pltpu.CompilerParams(
            use_tc_tiling_on_sc=True, needs_layout_passes=False
        ),
    )(_gather_kernel)
    return run(tbl2, idx_t)


def kernel(input_seq, embedding_table):
    tbl_t = embedding_table.T
    tail_t = jnp.pad(tbl_t[:, (_NTC - 1) * 128:], ((0, 0), (0, 64)))
    tbl2 = _transpose(tbl_t, tail_t)
    idx_t = input_seq.astype(jnp.int32).T
    out_t = _gather(tbl2, idx_t)
    return jnp.transpose(out_t, (2, 0, 1))


# speedup vs baseline: 2.9807x; 1.5546x over previous
import functools

import jax
import jax.numpy as jnp
from jax import lax
from jax.experimental import pallas as pl
from jax.experimental.pallas import tpu as pltpu
from jax.experimental.pallas import tpu_sc as plsc

_BATCH = 1024
_SEQ = 200
_EMBED = 64

_NC = 2
_NS = 16
_NW = _NC * _NS

_NBLK = _SEQ * (_BATCH // 128)
_BLK_PER_W = _NBLK // _NW
_NB = 3
_LOOK = 2
_N_MAIN = 48


def _gather_kernel(tbl_hbm, idx_hbm, out_hbm, idx_v, w_v, rows_v, outb_v,
                   isem, gsem, osem):
    wid = lax.axis_index("s") * _NC + lax.axis_index("c")
    c0 = wid * _BLK_PER_W
    iota16 = lax.iota(jnp.int32, 16)

    def blk_pos(j):
        c = c0 + j
        return c // 8, lax.rem(c, 8)

    for j in range(_BLK_PER_W):
        s, tb = blk_pos(j)
        pltpu.make_async_copy(
            idx_hbm.at[s, pl.ds(tb * 128, 128)], idx_v.at[j], isem
        ).start()
    for j in range(_BLK_PER_W):
        s, tb = blk_pos(j)
        pltpu.make_async_copy(
            idx_hbm.at[s, pl.ds(tb * 128, 128)], idx_v.at[j], isem
        ).wait()

    def prep_gather(j, slot):
        for g in range(8):
            v = idx_v[j, pl.ds(g * 16, 16)]
            w_v[slot, pl.ds(g * 16, 16)] = lax.shift_right_logical(v, 1)
        pltpu.make_async_copy(
            tbl_hbm.at[w_v.at[slot]], rows_v.at[slot], gsem.at[slot]
        ).start()

    def wait_gather(slot):
        pltpu.make_async_copy(
            tbl_hbm.at[w_v.at[slot]], rows_v.at[slot], gsem.at[slot]
        ).wait()

    def fire_out(j, slot):
        s, tb = blk_pos(j)
        pltpu.make_async_copy(
            outb_v.at[slot], out_hbm.at[s, :, pl.ds(tb * 128, 128)],
            osem.at[slot],
        ).start()

    def wait_out(j, slot):
        s, tb = blk_pos(j)
        pltpu.make_async_copy(
            outb_v.at[slot], out_hbm.at[s, :, pl.ds(tb * 128, 128)],
            osem.at[slot],
        ).wait()

    lane_vecs = [iota16 + g * 16 for g in range(8)]

    def transpose_block(j, slot):
        bases = []
        for g in range(8):
            v = idx_v[j, pl.ds(g * 16, 16)]
            bases.append(lax.shift_left(lax.bitwise_and(v, 1), 6))

        def ebody(e0, carry):
            e_vec = lax.bitwise_and(e0 + iota16, 63)
            for g in range(8):
                val = plsc.load_gather(
                    rows_v.at[slot], [lane_vecs[g], bases[g] + e_vec]
                )
                plsc.store_scatter(
                    outb_v.at[slot], [e_vec, lane_vecs[g]], val
                )
            return carry

        lax.fori_loop(0, _EMBED, ebody, 0, unroll=4)

    for k in range(_LOOK):
        prep_gather(k, k)

    def body(gidx, carry):
        for b in range(_NB):
            j = gidx * _NB + b
            wait_gather(b)

            @pl.when(j >= _NB)
            def _():
                wait_out(j - _NB, b)

            transpose_block(j, b)
            fire_out(j, b)

            @pl.when(j + _LOOK < _BLK_PER_W)
            def _():
                prep_gather(j + _LOOK, (b + _LOOK) % _NB)

        return carry

    lax.fori_loop(0, _N_MAIN // _NB, body, 0)

    for j in (48, 49):
        slot = j % _NB
        wait_gather(slot)
        wait_out(j - _NB, slot)
        transpose_block(j, slot)
        fire_out(j, slot)

    for j in (47, 48, 49):
        wait_out(j, j % _NB)


_NTC = 7813
_TC_PER_W = 245


def _transpose_kernel(tbl_hbm, tail_hbm, out_hbm, stg_v, tout_v, isem, osem):
    wid = lax.axis_index("s") * _NC + lax.axis_index("c")
    iota16 = lax.iota(jnp.int32, 16)

    def col_of(k):
        return wid + k * _NW

    def fire_in(k, slot):
        c = col_of(k)

        @pl.when(c < _NTC - 1)
        def _():
            pltpu.make_async_copy(
                tbl_hbm.at[:, pl.ds(c * 128, 128)], stg_v.at[slot],
                isem.at[slot],
            ).start()

        @pl.when(c == _NTC - 1)
        def _():
            pltpu.make_async_copy(
                tail_hbm, stg_v.at[slot], isem.at[slot]
            ).start()

    def wait_in(k, slot):
        c = col_of(k)

        @pl.when(c < _NTC - 1)
        def _():
            pltpu.make_async_copy(
                tbl_hbm.at[:, pl.ds(c * 128, 128)], stg_v.at[slot],
                isem.at[slot],
            ).wait()

        @pl.when(c == _NTC - 1)
        def _():
            pltpu.make_async_copy(
                tail_hbm, stg_v.at[slot], isem.at[slot]
            ).wait()

    lane_vecs = [iota16 + 16 * g for g in range(8)]

    def transpose_col(k, slot):
        def wbody(w0, carry):
            wr_vec = lax.bitwise_and(w0 + iota16, 63)
            col2 = wr_vec + wr_vec
            cols = [col2, col2 + 1]
            for g in range(8):
                val = plsc.load_gather(
                    stg_v.at[slot], [lane_vecs[g % 4], cols[g // 4]]
                )
                plsc.store_scatter(
                    tout_v.at[slot], [wr_vec, lane_vecs[g]], val
                )
            return carry

        lax.fori_loop(0, _EMBED, wbody, 0, unroll=4)

    def fire_out(k, slot):
        c = col_of(k)

        @pl.when(c < _NTC - 1)
        def _():
            pltpu.make_async_copy(
                tout_v.at[slot], out_hbm.at[pl.ds(c * 64, 64)], osem.at[slot]
            ).start()

        @pl.when(c == _NTC - 1)
        def _():
            pltpu.make_async_copy(
                tout_v.at[slot, pl.ds(0, 32)],
                out_hbm.at[pl.ds(c * 64, 32)], osem.at[slot],
            ).start()

    def wait_out(k, slot):
        c = col_of(k)

        @pl.when(c < _NTC - 1)
        def _():
            pltpu.make_async_copy(
                tout_v.at[slot], out_hbm.at[pl.ds(c * 64, 64)], osem.at[slot]
            ).wait()

        @pl.when(c == _NTC - 1)
        def _():
            pltpu.make_async_copy(
                tout_v.at[slot, pl.ds(0, 32)],
                out_hbm.at[pl.ds(c * 64, 32)], osem.at[slot],
            ).wait()

    def valid(k):
        return col_of(k) < _NTC

    def step(k, slot):
        @pl.when(valid(k))
        def _():
            wait_in(k, slot)

        @pl.when(valid(k + 1))
        def _():
            fire_in(k + 1, slot ^ 1)

        @pl.when(valid(k))
        def _():
            @pl.when(k >= 2)
            def _():
                wait_out(k - 2, slot)

            transpose_col(k, slot)
            fire_out(k, slot)

    @pl.when(valid(0))
    def _():
        fire_in(0, 0)

    def body(m, carry):
        step(2 * m, 0)
        step(2 * m + 1, 1)
        return carry

    lax.fori_loop(0, (_TC_PER_W - 1) // 2, body, 0)
    step(jnp.int32(_TC_PER_W - 1), (_TC_PER_W - 1) % 2)

    for k in (_TC_PER_W - 2, _TC_PER_W - 1):
        @pl.when(valid(k))
        def _():
            wait_out(k, k % 2)


@jax.jit
def _transpose(tbl_t, tail_t):
    mesh = plsc.VectorSubcoreMesh(core_axis_name="c", subcore_axis_name="s")
    run = functools.partial(
        pl.kernel,
        mesh=mesh,
        out_type=jax.ShapeDtypeStruct((500000, 128), jnp.float32),
        scratch_types=[
            pltpu.VMEM((2, _EMBED, 128), jnp.float32),
            pltpu.VMEM((2, _EMBED, 128), jnp.float32),
            pltpu.SemaphoreType.DMA((2,)),
            pltpu.SemaphoreType.DMA((2,)),
        ],
        compiler_params=pltpu.CompilerParams(
            use_tc_tiling_on_sc=True, needs_layout_passes=False
        ),
    )(_transpose_kernel)
    return run(tbl_t, tail_t)


@jax.jit
def _gather(tbl2, idx_t):
    mesh = plsc.VectorSubcoreMesh(core_axis_name="c", subcore_axis_name="s")
    run = functools.partial(
        pl.kernel,
        mesh=mesh,
        out_type=jax.ShapeDtypeStruct((_SEQ, _EMBED, _BATCH), jnp.float32),
        scratch_types=[
            pltpu.VMEM((_BLK_PER_W, 128), jnp.int32),
            pltpu.VMEM((_NB, 128), jnp.int32),
            pltpu.VMEM((_NB, 128, 128), jnp.float32),
            pltpu.VMEM((_NB, _EMBED, 128), jnp.float32),
            pltpu.SemaphoreType.DMA,
            pltpu.SemaphoreType.DMA((_NB,)),
            pltpu.SemaphoreType.DMA((_NB,)),
        ],
        compiler_params=pltpu.CompilerParams(
            use_tc_tiling_on_sc=True, needs_layout_passes=False
        ),
    )(_gather_kernel)
    return run(tbl2, idx_t)


def kernel(input_seq, embedding_table):
    tbl_t = embedding_table.T
    tail_t = jnp.pad(tbl_t[:, (_NTC - 1) * 128:], ((0, 0), (0, 64)))
    tbl2 = _transpose(tbl_t, tail_t)
    idx_t = input_seq.astype(jnp.int32).T
    out_t = _gather(tbl2, idx_t)
    return jnp.transpose(out_t, (2, 0, 1))


# speedup vs baseline: 4.8779x; 1.6365x over previous
import functools

import jax
import jax.numpy as jnp
from jax import lax
from jax.experimental import pallas as pl
from jax.experimental.pallas import tpu as pltpu
from jax.experimental.pallas import tpu_sc as plsc

_BATCH = 1024
_SEQ = 200
_EMBED = 64

_NC = 2
_NS = 16
_NW = _NC * _NS

_NBLK = _SEQ * (_BATCH // 128)
_BLK_PER_W = _NBLK // _NW
_NB = 3
_LOOK = 2
_N_MAIN = 48


def _gather_kernel(tbl_hbm, idx_hbm, out_hbm, idx_v, w_v, rows_v, outb_v,
                   isem, gsem, osem):
    wid = lax.axis_index("s") * _NC + lax.axis_index("c")
    c0 = wid * _BLK_PER_W
    iota16 = lax.iota(jnp.int32, 16)

    def blk_pos(j):
        c = c0 + j
        return c // 8, lax.rem(c, 8)

    for j in range(_BLK_PER_W):
        s, tb = blk_pos(j)
        pltpu.make_async_copy(
            idx_hbm.at[s, pl.ds(tb * 128, 128)], idx_v.at[j], isem
        ).start()
    for j in range(_BLK_PER_W):
        s, tb = blk_pos(j)
        pltpu.make_async_copy(
            idx_hbm.at[s, pl.ds(tb * 128, 128)], idx_v.at[j], isem
        ).wait()

    def prep_gather(j, slot):
        for g in range(8):
            v = idx_v[j, pl.ds(g * 16, 16)]
            w_v[slot, pl.ds(g * 16, 16)] = lax.shift_right_logical(v, 1)
        pltpu.make_async_copy(
            tbl_hbm.at[w_v.at[slot]], rows_v.at[slot], gsem.at[slot]
        ).start()

    def wait_gather(slot):
        pltpu.make_async_copy(
            tbl_hbm.at[w_v.at[slot]], rows_v.at[slot], gsem.at[slot]
        ).wait()

    def fire_out(j, slot):
        s, tb = blk_pos(j)
        pltpu.make_async_copy(
            outb_v.at[slot], out_hbm.at[s, :, pl.ds(tb * 128, 128)],
            osem.at[slot],
        ).start()

    def wait_out(j, slot):
        s, tb = blk_pos(j)
        pltpu.make_async_copy(
            outb_v.at[slot], out_hbm.at[s, :, pl.ds(tb * 128, 128)],
            osem.at[slot],
        ).wait()

    lane_vecs = [iota16 + g * 16 for g in range(8)]

    def transpose_block(j, slot):
        bases = []
        for g in range(8):
            v = idx_v[j, pl.ds(g * 16, 16)]
            bases.append(lax.shift_left(lax.bitwise_and(v, 1), 6))

        def gath(e0):
            e_vec = lax.bitwise_and(e0 + iota16, 63)
            vals = tuple(
                plsc.load_gather(
                    rows_v.at[slot], [lane_vecs[g], bases[g] + e_vec]
                )
                for g in range(8)
            )
            return vals, e_vec

        def ebody(e0, carry):
            prev_vals, prev_e = carry
            for g in range(8):
                plsc.store_scatter(
                    outb_v.at[slot], [prev_e, lane_vecs[g]], prev_vals[g]
                )
            return gath(lax.bitwise_and(e0, _EMBED - 1))

        lax.fori_loop(1, _EMBED + 1, ebody, gath(0), unroll=4)

    for k in range(_LOOK):
        prep_gather(k, k)

    def body(gidx, carry):
        for b in range(_NB):
            j = gidx * _NB + b
            wait_gather(b)

            @pl.when(j >= _NB)
            def _():
                wait_out(j - _NB, b)

            transpose_block(j, b)
            fire_out(j, b)

            @pl.when(j + _LOOK < _BLK_PER_W)
            def _():
                prep_gather(j + _LOOK, (b + _LOOK) % _NB)

        return carry

    lax.fori_loop(0, _N_MAIN // _NB, body, 0)

    for j in (48, 49):
        slot = j % _NB
        wait_gather(slot)
        wait_out(j - _NB, slot)
        transpose_block(j, slot)
        fire_out(j, slot)

    for j in (47, 48, 49):
        wait_out(j, j % _NB)


_NTC = 7813
_TC_PER_W = 245


def _transpose_kernel(tbl_hbm, tail_hbm, out_hbm, stg_v, tout_v, isem, osem):
    wid = lax.axis_index("s") * _NC + lax.axis_index("c")
    iota16 = lax.iota(jnp.int32, 16)

    def col_of(k):
        return wid + k * _NW

    def fire_in(k, slot):
        c = col_of(k)

        @pl.when(c < _NTC - 1)
        def _():
            pltpu.make_async_copy(
                tbl_hbm.at[:, pl.ds(c * 128, 128)], stg_v.at[slot],
                isem.at[slot],
            ).start()

        @pl.when(c == _NTC - 1)
        def _():
            pltpu.make_async_copy(
                tail_hbm, stg_v.at[slot], isem.at[slot]
            ).start()

    def wait_in(k, slot):
        c = col_of(k)

        @pl.when(c < _NTC - 1)
        def _():
            pltpu.make_async_copy(
                tbl_hbm.at[:, pl.ds(c * 128, 128)], stg_v.at[slot],
                isem.at[slot],
            ).wait()

        @pl.when(c == _NTC - 1)
        def _():
            pltpu.make_async_copy(
                tail_hbm, stg_v.at[slot], isem.at[slot]
            ).wait()

    lane_vecs = [iota16 + 16 * g for g in range(8)]

    def transpose_col(k, slot):
        def gath(w0):
            wr_vec = lax.bitwise_and(w0 + iota16, 63)
            col2 = wr_vec + wr_vec
            cols = [col2, col2 + 1]
            vals = tuple(
                plsc.load_gather(
                    stg_v.at[slot], [lane_vecs[g % 4], cols[g // 4]]
                )
                for g in range(8)
            )
            return vals, wr_vec

        def wbody(w0, carry):
            prev_vals, prev_wr = carry
            for g in range(8):
                plsc.store_scatter(
                    tout_v.at[slot], [prev_wr, lane_vecs[g]], prev_vals[g]
                )
            return gath(lax.bitwise_and(w0, _EMBED - 1))

        lax.fori_loop(1, _EMBED + 1, wbody, gath(0), unroll=4)

    def fire_out(k, slot):
        c = col_of(k)

        @pl.when(c < _NTC - 1)
        def _():
            pltpu.make_async_copy(
                tout_v.at[slot], out_hbm.at[pl.ds(c * 64, 64)], osem.at[slot]
            ).start()

        @pl.when(c == _NTC - 1)
        def _():
            pltpu.make_async_copy(
                tout_v.at[slot, pl.ds(0, 32)],
                out_hbm.at[pl.ds(c * 64, 32)], osem.at[slot],
            ).start()

    def wait_out(k, slot):
        c = col_of(k)

        @pl.when(c < _NTC - 1)
        def _():
            pltpu.make_async_copy(
                tout_v.at[slot], out_hbm.at[pl.ds(c * 64, 64)], osem.at[slot]
            ).wait()

        @pl.when(c == _NTC - 1)
        def _():
            pltpu.make_async_copy(
                tout_v.at[slot, pl.ds(0, 32)],
                out_hbm.at[pl.ds(c * 64, 32)], osem.at[slot],
            ).wait()

    def valid(k):
        return col_of(k) < _NTC

    def step(k, slot):
        @pl.when(valid(k))
        def _():
            wait_in(k, slot)

        @pl.when(valid(k + 1))
        def _():
            fire_in(k + 1, slot ^ 1)

        @pl.when(valid(k))
        def _():
            @pl.when(k >= 2)
            def _():
                wait_out(k - 2, slot)

            transpose_col(k, slot)
            fire_out(k, slot)

    @pl.when(valid(0))
    def _():
        fire_in(0, 0)

    def body(m, carry):
        step(2 * m, 0)
        step(2 * m + 1, 1)
        return carry

    lax.fori_loop(0, (_TC_PER_W - 1) // 2, body, 0)
    step(jnp.int32(_TC_PER_W - 1), (_TC_PER_W - 1) % 2)

    for k in (_TC_PER_W - 2, _TC_PER_W - 1):
        @pl.when(valid(k))
        def _():
            wait_out(k, k % 2)


@jax.jit
def _transpose(tbl_t, tail_t):
    mesh = plsc.VectorSubcoreMesh(core_axis_name="c", subcore_axis_name="s")
    run = functools.partial(
        pl.kernel,
        mesh=mesh,
        out_type=jax.ShapeDtypeStruct((500000, 128), jnp.float32),
        scratch_types=[
            pltpu.VMEM((2, _EMBED, 128), jnp.float32),
            pltpu.VMEM((2, _EMBED, 128), jnp.float32),
            pltpu.SemaphoreType.DMA((2,)),
            pltpu.SemaphoreType.DMA((2,)),
        ],
        compiler_params=pltpu.CompilerParams(
            use_tc_tiling_on_sc=True, needs_layout_passes=False
        ),
    )(_transpose_kernel)
    return run(tbl_t, tail_t)


@jax.jit
def _gather(tbl2, idx_t):
    mesh = plsc.VectorSubcoreMesh(core_axis_name="c", subcore_axis_name="s")
    run = functools.partial(
        pl.kernel,
        mesh=mesh,
        out_type=jax.ShapeDtypeStruct((_SEQ, _EMBED, _BATCH), jnp.float32),
        scratch_types=[
            pltpu.VMEM((_BLK_PER_W, 128), jnp.int32),
            pltpu.VMEM((_NB, 128), jnp.int32),
            pltpu.VMEM((_NB, 128, 128), jnp.float32),
            pltpu.VMEM((_NB, _EMBED, 128), jnp.float32),
            pltpu.SemaphoreType.DMA,
            pltpu.SemaphoreType.DMA((_NB,)),
            pltpu.SemaphoreType.DMA((_NB,)),
        ],
        compiler_params=pltpu.CompilerParams(
            use_tc_tiling_on_sc=True, needs_layout_passes=False
        ),
    )(_gather_kernel)
    return run(tbl2, idx_t)


def kernel(input_seq, embedding_table):
    tbl_t = embedding_table.T
    tail_t = jnp.pad(tbl_t[:, (_NTC - 1) * 128:], ((0, 0), (0, 64)))
    tbl2 = _transpose(tbl_t, tail_t)
    idx_t = input_seq.astype(jnp.int32).T
    out_t = _gather(tbl2, idx_t)
    return jnp.transpose(out_t, (2, 0, 1))
